# baseline (speedup 1.0000x reference)
import functools

import jax
import jax.numpy as jnp
from jax import lax
from jax.experimental import pallas as pl
from jax.experimental.pallas import tpu as pltpu
from jax.experimental.pallas import tpu_sc as plsc

R = 128
N = 32768
K = 256
L = 16
NC = 2
NS = 16
NW = NC * NS
ROWS_PER_W = R // NW
NCHUNK = N // L


def _sc_body(x_hbm, okey_hbm, oidx_hbm, rowbuf, cand, hist, hist2, okey,
             oidx):
    wid = lax.axis_index("s") * NC + lax.axis_index("c")
    lane = lax.iota(jnp.int32, L)
    lane256 = lane * jnp.int32(256)
    ones = jnp.ones((L,), jnp.int32)
    zeros = jnp.zeros((L,), jnp.int32)
    full = jnp.ones((L,), jnp.bool_)

    def zero_hist():
        for g in range(256 // L):
            hist[pl.ds(g * L, L)] = zeros

    def hist_at(b):
        return hist[pl.ds(b, L)][0]

    def find_bucket(need):
        def cond(st):
            b, acc = st
            return acc < need

        def body(st):
            b, acc = st
            b = b - 1
            return b, acc + hist_at(b)

        b, acc = lax.while_loop(cond, body, (jnp.int32(256), jnp.int32(0)))
        return b, acc - hist_at(b)

    def do_row(r, _):
        row = wid * ROWS_PER_W + r
        pltpu.sync_copy(x_hbm.at[row], rowbuf)

        @plsc.parallel_loop(0, 256, unroll=8)
        def _zero2(i):
            hist2[pl.ds(i * L, L)] = zeros

        @plsc.parallel_loop(0, NCHUNK, unroll=8)
        def _pass1(i):
            s = rowbuf[pl.ds(i * L, L)]
            m = s >> 31
            ikey = s ^ (m & jnp.int32(0x7FFFFFFF))
            rowbuf[pl.ds(i * L, L)] = ikey
            bins = (ikey >> 24) + jnp.int32(128)
            plsc.addupdate_scatter(hist2, [lane256 + bins], ones, mask=full)

        for g in range(256 // L):
            t = hist2[pl.ds(g * L, L)]
            for l in range(1, L):
                t = t + hist2[pl.ds(l * 256 + g * L, L)]
            hist[pl.ds(g * L, L)] = t

        b1, above1 = find_bucket(jnp.int32(K))

        @plsc.parallel_loop(0, NCHUNK, unroll=4, carry=(zeros, zeros))
        def _pass2(i, st):
            wtop_v, wc_v = st
            ikey = rowbuf[pl.ds(i * L, L)]
            idxv = i * L + lane
            bins = (ikey >> 24) + jnp.int32(128)
            m_top = bins > b1
            m_eq = bins == b1
            pf_t = plsc.cumsum(m_top.astype(jnp.int32))
            pos_t = wtop_v + pf_t - 1
            plsc.store_scatter(okey, [pos_t], ikey, mask=m_top)
            plsc.store_scatter(oidx, [pos_t], idxv, mask=m_top)
            wtop_v = wtop_v + plsc.all_reduce_population_count(m_top)
            pf_e = plsc.cumsum(m_eq.astype(jnp.int32))
            pos_e = wc_v + pf_e - 1
            plsc.store_scatter(cand, [pos_e], idxv, mask=m_eq)
            wc_v = wc_v + plsc.all_reduce_population_count(m_eq)
            return wtop_v, wc_v

        wtop_v, wc_v = _pass2
        wtop = wtop_v[0]
        c = wc_v[0]
        need = jnp.int32(K) - wtop

        for shift in (16, 8, 0):
            zero_hist()
            nv = (c + (L - 1)) >> 4

            @plsc.parallel_loop(0, nv, unroll=2)
            def _histbody(i):
                idxv = cand[pl.ds(i * L, L)]
                valid = (i * L + lane) < c
                keyv = plsc.load_gather(rowbuf, [idxv], mask=valid)
                ub = (keyv >> shift) & jnp.int32(0xFF)
                plsc.addupdate_scatter(hist, [ub], ones, mask=valid)

            b2, above2 = find_bucket(need)

            @plsc.parallel_loop(0, nv, unroll=2,
                                carry=(jnp.full((L,), wtop, jnp.int32),
                                       zeros))
            def _appbody(i, st):
                wtop_v, wc_v = st
                idxv = cand[pl.ds(i * L, L)]
                valid = (i * L + lane) < c
                keyv = plsc.load_gather(rowbuf, [idxv], mask=valid)
                ub = (keyv >> shift) & jnp.int32(0xFF)
                m_top = valid & (ub > b2)
                m_eq = valid & (ub == b2)
                pf_t = plsc.cumsum(m_top.astype(jnp.int32))
                pos_t = wtop_v + pf_t - 1
                plsc.store_scatter(okey, [pos_t], keyv, mask=m_top)
                plsc.store_scatter(oidx, [pos_t], idxv, mask=m_top)
                wtop_v = wtop_v + plsc.all_reduce_population_count(m_top)
                pf_e = plsc.cumsum(m_eq.astype(jnp.int32))
                pos_e = wc_v + pf_e - 1
                plsc.store_scatter(cand, [pos_e], idxv, mask=m_eq)
                wc_v = wc_v + plsc.all_reduce_population_count(m_eq)
                return wtop_v, wc_v

            wtop_v, wc_v = _appbody
            wtop = wtop_v[0]
            c = wc_v[0]
            need = need - above2

        nv = (c + (L - 1)) >> 4

        def finbody(i, st):
            wtop, rem = st
            idxv = cand[pl.ds(i * L, L)]
            valid = (i * L + lane) < c
            pc = plsc.cumsum(valid.astype(jnp.int32))
            m = valid & (pc <= rem)
            keyv = plsc.load_gather(rowbuf, [idxv], mask=m)
            plsc.store_compressed(okey.at[pl.ds(wtop, L)], keyv, mask=m)
            plsc.store_compressed(oidx.at[pl.ds(wtop, L)], idxv, mask=m)
            cnt = jnp.sum(m.astype(jnp.int32))
            return wtop + cnt, rem - cnt

        lax.fori_loop(0, nv, finbody, (wtop, need))

        pltpu.sync_copy(okey.at[pl.ds(0, K)], okey_hbm.at[row])
        pltpu.sync_copy(oidx.at[pl.ds(0, K)], oidx_hbm.at[row])
        return _

    lax.fori_loop(0, ROWS_PER_W, do_row, jnp.int32(0))


def _sc_select(xb):
    mesh = plsc.VectorSubcoreMesh(core_axis_name="c", subcore_axis_name="s")
    f = pl.kernel(
        _sc_body,
        out_type=(
            jax.ShapeDtypeStruct((R, K), jnp.int32),
            jax.ShapeDtypeStruct((R, K), jnp.int32),
        ),
        mesh=mesh,
        compiler_params=pltpu.CompilerParams(needs_layout_passes=False),
        scratch_types=[
            pltpu.VMEM((N,), jnp.int32),
            pltpu.VMEM((N + L,), jnp.int32),
            pltpu.VMEM((256 + L,), jnp.int32),
            pltpu.VMEM((256 * L,), jnp.int32),
            pltpu.VMEM((K + L,), jnp.int32),
            pltpu.VMEM((K + L,), jnp.int32),
        ],
    )
    return f(xb)


def _roll_xor(x, pos, stride):
    bit = (pos & stride) == 0
    return jnp.where(bit, jnp.roll(x, -stride, axis=1),
                     jnp.roll(x, stride, axis=1))


def _tc_sort_body(key_ref, idx_ref, val_ref, ind_ref):
    keys = key_ref[...]
    idxs = idx_ref[...]
    pos = lax.broadcasted_iota(jnp.int32, (R, K), 1)
    size = 2
    while size <= K:
        stride = size // 2
        while stride >= 1:
            pk = _roll_xor(keys, pos, stride)
            pi = _roll_xor(idxs, pos, stride)
            a_pre_b = (keys > pk) | ((keys == pk) & (idxs < pi))
            second = (pos & stride) != 0
            flip = (pos & size) != 0
            take_a = a_pre_b ^ second ^ flip
            keys = jnp.where(take_a, keys, pk)
            idxs = jnp.where(take_a, idxs, pi)
            stride //= 2
        size *= 2
    s = jnp.where(keys >= 0, keys, keys ^ jnp.int32(0x7FFFFFFF))
    val_ref[...] = lax.bitcast_convert_type(s, jnp.float32)
    ind_ref[...] = idxs


def _tc_sort(okey, oidx):
    return pl.pallas_call(
        _tc_sort_body,
        out_shape=(
            jax.ShapeDtypeStruct((R, K), jnp.float32),
            jax.ShapeDtypeStruct((R, K), jnp.int32),
        ),
    )(okey, oidx)


def kernel(x, k):
    xb = lax.bitcast_convert_type(x, jnp.int32)
    okey, oidx = _sc_select(xb)
    vals, inds = _tc_sort(okey, oidx)
    vals = vals + (jnp.asarray(k) - K).astype(vals.dtype)
    return vals, inds


# speedup vs baseline: 1.0141x; 1.0141x over previous
import functools

import jax
import jax.numpy as jnp
from jax import lax
from jax.experimental import pallas as pl
from jax.experimental.pallas import tpu as pltpu
from jax.experimental.pallas import tpu_sc as plsc

R = 128
N = 32768
K = 256
L = 16
NC = 2
NS = 16
NW = NC * NS
ROWS_PER_W = R // NW
NCHUNK = N // L


def _sc_body(x_hbm, okey_hbm, oidx_hbm, rowbuf, cand, hist, okey, oidx):
    wid = lax.axis_index("s") * NC + lax.axis_index("c")
    lane = lax.iota(jnp.int32, L)
    lane256 = lane * jnp.int32(256)
    ones = jnp.ones((L,), jnp.int32)
    zeros = jnp.zeros((L,), jnp.int32)
    full = jnp.ones((L,), jnp.bool_)

    def zero_hist():
        for g in range(256 // L):
            hist[pl.ds(g * L, L)] = zeros

    def hist_at(b):
        return hist[pl.ds(b, L)][0]

    def find_bucket(need):
        def cond(st):
            b, acc = st
            return acc < need

        def body(st):
            b, acc = st
            b = b - 1
            return b, acc + hist_at(b)

        b, acc = lax.while_loop(cond, body, (jnp.int32(256), jnp.int32(0)))
        return b, acc - hist_at(b)

    def do_row(r, _):
        row = wid * ROWS_PER_W + r
        pltpu.sync_copy(x_hbm.at[row], rowbuf)

        zero_hist()

        @plsc.parallel_loop(0, NCHUNK, unroll=8)
        def _pass1(i):
            s = rowbuf[pl.ds(i * L, L)]
            m = s >> 31
            ikey = s ^ (m & jnp.int32(0x7FFFFFFF))
            rowbuf[pl.ds(i * L, L)] = ikey
            bins = (ikey >> 24) + jnp.int32(128)
            plsc.addupdate_scatter(hist, [bins], ones, mask=full)

        b1, above1 = find_bucket(jnp.int32(K))

        @plsc.parallel_loop(0, NCHUNK, unroll=4, carry=(zeros, zeros))
        def _pass2(i, st):
            wtop_v, wc_v = st
            ikey = rowbuf[pl.ds(i * L, L)]
            idxv = i * L + lane
            bins = (ikey >> 24) + jnp.int32(128)
            m_top = bins > b1
            m_eq = bins == b1
            pf_t = plsc.cumsum(m_top.astype(jnp.int32))
            pos_t = wtop_v + pf_t - 1
            plsc.store_scatter(okey, [pos_t], ikey, mask=m_top)
            plsc.store_scatter(oidx, [pos_t], idxv, mask=m_top)
            wtop_v = wtop_v + plsc.all_reduce_population_count(m_top)
            pf_e = plsc.cumsum(m_eq.astype(jnp.int32))
            pos_e = wc_v + pf_e - 1
            plsc.store_scatter(cand, [pos_e], idxv, mask=m_eq)
            wc_v = wc_v + plsc.all_reduce_population_count(m_eq)
            return wtop_v, wc_v

        wtop_v, wc_v = _pass2
        wtop = wtop_v[0]
        c = wc_v[0]
        need = jnp.int32(K) - wtop

        for shift in (16, 8, 0):
            zero_hist()
            nv = (c + (L - 1)) >> 4

            @plsc.parallel_loop(0, nv, unroll=2)
            def _histbody(i):
                idxv = cand[pl.ds(i * L, L)]
                valid = (i * L + lane) < c
                keyv = plsc.load_gather(rowbuf, [idxv], mask=valid)
                ub = (keyv >> shift) & jnp.int32(0xFF)
                plsc.addupdate_scatter(hist, [ub], ones, mask=valid)

            b2, above2 = find_bucket(need)

            @plsc.parallel_loop(0, nv, unroll=2,
                                carry=(jnp.full((L,), wtop, jnp.int32),
                                       zeros))
            def _appbody(i, st):
                wtop_v, wc_v = st
                idxv = cand[pl.ds(i * L, L)]
                valid = (i * L + lane) < c
                keyv = plsc.load_gather(rowbuf, [idxv], mask=valid)
                ub = (keyv >> shift) & jnp.int32(0xFF)
                m_top = valid & (ub > b2)
                m_eq = valid & (ub == b2)
                pf_t = plsc.cumsum(m_top.astype(jnp.int32))
                pos_t = wtop_v + pf_t - 1
                plsc.store_scatter(okey, [pos_t], keyv, mask=m_top)
                plsc.store_scatter(oidx, [pos_t], idxv, mask=m_top)
                wtop_v = wtop_v + plsc.all_reduce_population_count(m_top)
                pf_e = plsc.cumsum(m_eq.astype(jnp.int32))
                pos_e = wc_v + pf_e - 1
                plsc.store_scatter(cand, [pos_e], idxv, mask=m_eq)
                wc_v = wc_v + plsc.all_reduce_population_count(m_eq)
                return wtop_v, wc_v

            wtop_v, wc_v = _appbody
            wtop = wtop_v[0]
            c = wc_v[0]
            need = need - above2

        nv = (c + (L - 1)) >> 4

        def finbody(i, st):
            wtop, rem = st
            idxv = cand[pl.ds(i * L, L)]
            valid = (i * L + lane) < c
            pc = plsc.cumsum(valid.astype(jnp.int32))
            m = valid & (pc <= rem)
            keyv = plsc.load_gather(rowbuf, [idxv], mask=m)
            plsc.store_compressed(okey.at[pl.ds(wtop, L)], keyv, mask=m)
            plsc.store_compressed(oidx.at[pl.ds(wtop, L)], idxv, mask=m)
            cnt = jnp.sum(m.astype(jnp.int32))
            return wtop + cnt, rem - cnt

        lax.fori_loop(0, nv, finbody, (wtop, need))

        pltpu.sync_copy(okey.at[pl.ds(0, K)], okey_hbm.at[row])
        pltpu.sync_copy(oidx.at[pl.ds(0, K)], oidx_hbm.at[row])
        return _

    lax.fori_loop(0, ROWS_PER_W, do_row, jnp.int32(0))


def _sc_select(xb):
    mesh = plsc.VectorSubcoreMesh(core_axis_name="c", subcore_axis_name="s")
    f = pl.kernel(
        _sc_body,
        out_type=(
            jax.ShapeDtypeStruct((R, K), jnp.int32),
            jax.ShapeDtypeStruct((R, K), jnp.int32),
        ),
        mesh=mesh,
        compiler_params=pltpu.CompilerParams(needs_layout_passes=False),
        scratch_types=[
            pltpu.VMEM((N,), jnp.int32),
            pltpu.VMEM((N + L,), jnp.int32),
            pltpu.VMEM((256 + L,), jnp.int32),
            pltpu.VMEM((K + L,), jnp.int32),
            pltpu.VMEM((K + L,), jnp.int32),
        ],
    )
    return f(xb)


def _roll_xor(x, pos, stride):
    bit = (pos & stride) == 0
    return jnp.where(bit, jnp.roll(x, -stride, axis=1),
                     jnp.roll(x, stride, axis=1))


def _tc_sort_body(key_ref, idx_ref, val_ref, ind_ref):
    keys = key_ref[...]
    idxs = idx_ref[...]
    pos = lax.broadcasted_iota(jnp.int32, (R, K), 1)
    size = 2
    while size <= K:
        stride = size // 2
        while stride >= 1:
            pk = _roll_xor(keys, pos, stride)
            pi = _roll_xor(idxs, pos, stride)
            a_pre_b = (keys > pk) | ((keys == pk) & (idxs < pi))
            second = (pos & stride) != 0
            flip = (pos & size) != 0
            take_a = a_pre_b ^ second ^ flip
            keys = jnp.where(take_a, keys, pk)
            idxs = jnp.where(take_a, idxs, pi)
            stride //= 2
        size *= 2
    s = jnp.where(keys >= 0, keys, keys ^ jnp.int32(0x7FFFFFFF))
    val_ref[...] = lax.bitcast_convert_type(s, jnp.float32)
    ind_ref[...] = idxs


def _tc_sort(okey, oidx):
    return pl.pallas_call(
        _tc_sort_body,
        out_shape=(
            jax.ShapeDtypeStruct((R, K), jnp.float32),
            jax.ShapeDtypeStruct((R, K), jnp.int32),
        ),
    )(okey, oidx)


def kernel(x, k):
    xb = lax.bitcast_convert_type(x, jnp.int32)
    okey, oidx = _sc_select(xb)
    vals, inds = _tc_sort(okey, oidx)
    vals = vals + (jnp.asarray(k) - K).astype(vals.dtype)
    return vals, inds


# speedup vs baseline: 1.0393x; 1.0249x over previous
import functools

import jax
import jax.numpy as jnp
from jax import lax
from jax.experimental import pallas as pl
from jax.experimental.pallas import tpu as pltpu
from jax.experimental.pallas import tpu_sc as plsc

R = 128
N = 32768
K = 256
L = 16
NC = 2
NS = 16
NW = NC * NS
ROWS_PER_W = R // NW
NCHUNK = N // L


def _sc_body(x_hbm, okey_hbm, oidx_hbm, rowbuf0, rowbuf1, cand, hist, okey,
             oidx, sem0, sem1):
    wid = lax.axis_index("s") * NC + lax.axis_index("c")
    lane = lax.iota(jnp.int32, L)
    lane256 = lane * jnp.int32(256)
    ones = jnp.ones((L,), jnp.int32)
    zeros = jnp.zeros((L,), jnp.int32)
    full = jnp.ones((L,), jnp.bool_)

    def zero_hist():
        for g in range(256 // L):
            hist[pl.ds(g * L, L)] = zeros

    def hist_at(b):
        return hist[pl.ds(b, L)][0]

    def find_bucket(need):
        def cond(st):
            b, acc = st
            return acc < need

        def body(st):
            b, acc = st
            b = b - 1
            return b, acc + hist_at(b)

        b, acc = lax.while_loop(cond, body, (jnp.int32(256), jnp.int32(0)))
        return b, acc - hist_at(b)

    def do_row(rowbuf, row):

        zero_hist()

        @plsc.parallel_loop(0, NCHUNK, unroll=8)
        def _pass1(i):
            s = rowbuf[pl.ds(i * L, L)]
            m = s >> 31
            ikey = s ^ (m & jnp.int32(0x7FFFFFFF))
            rowbuf[pl.ds(i * L, L)] = ikey
            bins = (ikey >> 24) + jnp.int32(128)
            plsc.addupdate_scatter(hist, [bins], ones, mask=full)

        b1, above1 = find_bucket(jnp.int32(K))

        @plsc.parallel_loop(0, NCHUNK, unroll=4, carry=(zeros, zeros))
        def _pass2(i, st):
            wtop_v, wc_v = st
            ikey = rowbuf[pl.ds(i * L, L)]
            idxv = i * L + lane
            bins = (ikey >> 24) + jnp.int32(128)
            m_top = bins > b1
            m_eq = bins == b1
            pf_t = plsc.cumsum(ones, mask=m_top)
            pos_t = wtop_v + pf_t - 1
            plsc.store_scatter(okey, [pos_t], ikey, mask=m_top)
            plsc.store_scatter(oidx, [pos_t], idxv, mask=m_top)
            wtop_v = wtop_v + plsc.all_reduce_population_count(m_top)
            pf_e = plsc.cumsum(ones, mask=m_eq)
            pos_e = wc_v + pf_e - 1
            plsc.store_scatter(cand, [pos_e], idxv, mask=m_eq)
            wc_v = wc_v + plsc.all_reduce_population_count(m_eq)
            return wtop_v, wc_v

        wtop_v, wc_v = _pass2
        wtop = wtop_v[0]
        c = wc_v[0]
        need = jnp.int32(K) - wtop

        for shift in (16, 8, 0):
            zero_hist()
            nv = (c + (L - 1)) >> 4

            @plsc.parallel_loop(0, nv, unroll=2)
            def _histbody(i):
                idxv = cand[pl.ds(i * L, L)]
                valid = (i * L + lane) < c
                keyv = plsc.load_gather(rowbuf, [idxv], mask=valid)
                ub = (keyv >> shift) & jnp.int32(0xFF)
                plsc.addupdate_scatter(hist, [ub], ones, mask=valid)

            b2, above2 = find_bucket(need)

            @plsc.parallel_loop(0, nv, unroll=2,
                                carry=(jnp.full((L,), wtop, jnp.int32),
                                       zeros))
            def _appbody(i, st):
                wtop_v, wc_v = st
                idxv = cand[pl.ds(i * L, L)]
                valid = (i * L + lane) < c
                keyv = plsc.load_gather(rowbuf, [idxv], mask=valid)
                ub = (keyv >> shift) & jnp.int32(0xFF)
                m_top = valid & (ub > b2)
                m_eq = valid & (ub == b2)
                pf_t = plsc.cumsum(ones, mask=m_top)
                pos_t = wtop_v + pf_t - 1
                plsc.store_scatter(okey, [pos_t], keyv, mask=m_top)
                plsc.store_scatter(oidx, [pos_t], idxv, mask=m_top)
                wtop_v = wtop_v + plsc.all_reduce_population_count(m_top)
                pf_e = plsc.cumsum(ones, mask=m_eq)
                pos_e = wc_v + pf_e - 1
                plsc.store_scatter(cand, [pos_e], idxv, mask=m_eq)
                wc_v = wc_v + plsc.all_reduce_population_count(m_eq)
                return wtop_v, wc_v

            wtop_v, wc_v = _appbody
            wtop = wtop_v[0]
            c = wc_v[0]
            need = need - above2

        nv = (c + (L - 1)) >> 4

        def finbody(i, st):
            wtop, rem = st
            idxv = cand[pl.ds(i * L, L)]
            valid = (i * L + lane) < c
            pc = plsc.cumsum(valid.astype(jnp.int32))
            m = valid & (pc <= rem)
            keyv = plsc.load_gather(rowbuf, [idxv], mask=m)
            plsc.store_compressed(okey.at[pl.ds(wtop, L)], keyv, mask=m)
            plsc.store_compressed(oidx.at[pl.ds(wtop, L)], idxv, mask=m)
            cnt = jnp.sum(m.astype(jnp.int32))
            return wtop + cnt, rem - cnt

        lax.fori_loop(0, nv, finbody, (wtop, need))

        pltpu.sync_copy(okey.at[pl.ds(0, K)], okey_hbm.at[row])
        pltpu.sync_copy(oidx.at[pl.ds(0, K)], oidx_hbm.at[row])

    base = wid * ROWS_PER_W
    bufs = (rowbuf0, rowbuf1)
    sems = (sem0, sem1)
    handles = [pltpu.async_copy(x_hbm.at[base], rowbuf0, sem0), None]
    for r in range(ROWS_PER_W):
        if r + 1 < ROWS_PER_W:
            nb = (r + 1) % 2
            handles[nb] = pltpu.async_copy(x_hbm.at[base + (r + 1)],
                                           bufs[nb], sems[nb])
        handles[r % 2].wait()
        do_row(bufs[r % 2], base + r)


def _sc_select(xb):
    mesh = plsc.VectorSubcoreMesh(core_axis_name="c", subcore_axis_name="s")
    f = pl.kernel(
        _sc_body,
        out_type=(
            jax.ShapeDtypeStruct((R, K), jnp.int32),
            jax.ShapeDtypeStruct((R, K), jnp.int32),
        ),
        mesh=mesh,
        compiler_params=pltpu.CompilerParams(needs_layout_passes=False),
        scratch_types=[
            pltpu.VMEM((N,), jnp.int32),
            pltpu.VMEM((N,), jnp.int32),
            pltpu.VMEM((N + L,), jnp.int32),
            pltpu.VMEM((256 + L,), jnp.int32),
            pltpu.VMEM((K + L,), jnp.int32),
            pltpu.VMEM((K + L,), jnp.int32),
            pltpu.SemaphoreType.DMA,
            pltpu.SemaphoreType.DMA,
        ],
    )
    return f(xb)


def _roll_xor(x, pos, stride):
    bit = (pos & stride) == 0
    return jnp.where(bit, jnp.roll(x, -stride, axis=1),
                     jnp.roll(x, stride, axis=1))


def _tc_sort_body(key_ref, idx_ref, val_ref, ind_ref):
    keys = key_ref[...]
    idxs = idx_ref[...]
    pos = lax.broadcasted_iota(jnp.int32, (R, K), 1)
    size = 2
    while size <= K:
        stride = size // 2
        while stride >= 1:
            pk = _roll_xor(keys, pos, stride)
            pi = _roll_xor(idxs, pos, stride)
            a_pre_b = (keys > pk) | ((keys == pk) & (idxs < pi))
            second = (pos & stride) != 0
            flip = (pos & size) != 0
            take_a = a_pre_b ^ second ^ flip
            keys = jnp.where(take_a, keys, pk)
            idxs = jnp.where(take_a, idxs, pi)
            stride //= 2
        size *= 2
    s = jnp.where(keys >= 0, keys, keys ^ jnp.int32(0x7FFFFFFF))
    val_ref[...] = lax.bitcast_convert_type(s, jnp.float32)
    ind_ref[...] = idxs


def _tc_sort(okey, oidx):
    return pl.pallas_call(
        _tc_sort_body,
        out_shape=(
            jax.ShapeDtypeStruct((R, K), jnp.float32),
            jax.ShapeDtypeStruct((R, K), jnp.int32),
        ),
    )(okey, oidx)


def kernel(x, k):
    xb = lax.bitcast_convert_type(x, jnp.int32)
    okey, oidx = _sc_select(xb)
    vals, inds = _tc_sort(okey, oidx)
    vals = vals + (jnp.asarray(k) - K).astype(vals.dtype)
    return vals, inds


# speedup vs baseline: 1.4860x; 1.4298x over previous
import functools

import jax
import jax.numpy as jnp
from jax import lax
from jax.experimental import pallas as pl
from jax.experimental.pallas import tpu as pltpu
from jax.experimental.pallas import tpu_sc as plsc

R = 128
N = 32768
K = 256
L = 16
NC = 2
NS = 16
NW = NC * NS
ROWS_PER_W = R // NW
NCHUNK = N // L


def _sc_body(x_hbm, okey_hbm, oidx_hbm, rowbuf0, rowbuf1, cand, hist, okey,
             oidx, sem0, sem1):
    wid = lax.axis_index("s") * NC + lax.axis_index("c")
    lane = lax.iota(jnp.int32, L)
    ones = jnp.ones((L,), jnp.int32)
    zeros = jnp.zeros((L,), jnp.int32)
    full = jnp.ones((L,), jnp.bool_)

    def zero_hist():
        for g in range(256 // L):
            hist[pl.ds(g * L, L)] = zeros

    def find_bucket(need):
        sums = [jnp.sum(hist[pl.ds(g * L, L)]) for g in range(256 // L)]
        sfx = [jnp.int32(0)] * 17
        for g in range(15, -1, -1):
            sfx[g] = sfx[g + 1] + sums[g]
        cnt_g = jnp.int32(0)
        for g in range(16):
            cnt_g = cnt_g + (sfx[g] >= need).astype(jnp.int32)
        gsel = cnt_g - 1
        above_g = jnp.int32(0)
        for g in range(16):
            above_g = jnp.where(gsel == g, sfx[g + 1], above_g)
        h = hist[pl.ds(gsel * L, L)]
        sfx_in = lax.rev(plsc.cumsum(lax.rev(h, (0,))), (0,)) + above_g
        m = sfx_in >= need
        cnt_v = plsc.all_reduce_population_count(m)
        b = gsel * L + cnt_v[0] - 1
        above = jnp.sum(jnp.where(lane >= cnt_v, h, 0)) + above_g
        return b, above

    def do_row(rowbuf, row):
        zero_hist()

        @plsc.parallel_loop(0, NCHUNK, unroll=8)
        def _pass1(i):
            s = rowbuf[pl.ds(i * L, L)]
            m = s >> 31
            ikey = s ^ (m & jnp.int32(0x7FFFFFFF))
            rowbuf[pl.ds(i * L, L)] = ikey
            bins = (ikey >> 24) + jnp.int32(128)
            plsc.addupdate_scatter(hist, [bins], ones, mask=full)

        b1, above1 = find_bucket(jnp.int32(K))

        @plsc.parallel_loop(0, NCHUNK, unroll=4, carry=zeros)
        def _pass2(i, wc_v):
            ikey = rowbuf[pl.ds(i * L, L)]
            bins = (ikey >> 24) + jnp.int32(128)
            m = bins >= b1
            pf = plsc.cumsum(ones, mask=m)
            plsc.store_scatter(cand, [wc_v + pf - 1], i * L + lane, mask=m)
            return wc_v + plsc.all_reduce_population_count(m)

        c = _pass2[0]
        wtop = jnp.int32(0)
        need = jnp.int32(K)

        for shift in (24, 16, 8, 0):
            if shift == 24:
                b2, above2 = b1, above1
            else:
                zero_hist()
                nv_h = (c + (L - 1)) >> 4

                @plsc.parallel_loop(0, nv_h, unroll=2)
                def _histbody(i):
                    idxv = cand[pl.ds(i * L, L)]
                    valid = (i * L + lane) < c
                    keyv = plsc.load_gather(rowbuf, [idxv], mask=valid)
                    ub = (keyv >> shift) & jnp.int32(0xFF)
                    plsc.addupdate_scatter(hist, [ub], ones, mask=valid)

                b2, above2 = find_bucket(need)

            nv = (c + (L - 1)) >> 4

            @plsc.parallel_loop(0, nv, unroll=2,
                                carry=(jnp.full((L,), wtop, jnp.int32),
                                       zeros))
            def _appbody(i, st):
                wtop_v, wc_v = st
                idxv = cand[pl.ds(i * L, L)]
                valid = (i * L + lane) < c
                keyv = plsc.load_gather(rowbuf, [idxv], mask=valid)
                if shift == 24:
                    ub = (keyv >> 24) + jnp.int32(128)
                else:
                    ub = (keyv >> shift) & jnp.int32(0xFF)
                m_top = valid & (ub > b2)
                m_eq = valid & (ub == b2)
                pf_t = plsc.cumsum(ones, mask=m_top)
                plsc.store_scatter(oidx, [wtop_v + pf_t - 1], idxv,
                                   mask=m_top)
                wtop_v = wtop_v + plsc.all_reduce_population_count(m_top)
                pf_e = plsc.cumsum(ones, mask=m_eq)
                plsc.store_scatter(cand, [wc_v + pf_e - 1], idxv, mask=m_eq)
                wc_v = wc_v + plsc.all_reduce_population_count(m_eq)
                return wtop_v, wc_v

            wtop_v, wc_v = _appbody
            wtop = wtop_v[0]
            c = wc_v[0]
            need = need - above2

        nv = (c + (L - 1)) >> 4

        @plsc.parallel_loop(0, nv, unroll=2,
                            carry=(jnp.full((L,), wtop, jnp.int32),
                                   jnp.full((L,), need, jnp.int32)))
        def _finbody(i, st):
            wtop_v, rem_v = st
            idxv = cand[pl.ds(i * L, L)]
            valid = (i * L + lane) < c
            pc = plsc.cumsum(ones, mask=valid)
            m = valid & (pc <= rem_v)
            plsc.store_scatter(oidx, [wtop_v + pc - 1], idxv, mask=m)
            cnt = plsc.all_reduce_population_count(m)
            return wtop_v + cnt, rem_v - cnt

        _finbody

        for j in range(K // L):
            idxv = oidx[pl.ds(j * L, L)]
            okey[pl.ds(j * L, L)] = plsc.load_gather(rowbuf, [idxv],
                                                     mask=full)

        pltpu.sync_copy(okey.at[pl.ds(0, K)], okey_hbm.at[row])
        pltpu.sync_copy(oidx.at[pl.ds(0, K)], oidx_hbm.at[row])

    base = wid * ROWS_PER_W
    bufs = (rowbuf0, rowbuf1)
    sems = (sem0, sem1)
    handles = [pltpu.async_copy(x_hbm.at[base], rowbuf0, sem0), None]
    for r in range(ROWS_PER_W):
        if r + 1 < ROWS_PER_W:
            nb = (r + 1) % 2
            handles[nb] = pltpu.async_copy(x_hbm.at[base + (r + 1)],
                                           bufs[nb], sems[nb])
        handles[r % 2].wait()
        do_row(bufs[r % 2], base + r)


def _sc_select(xb):
    mesh = plsc.VectorSubcoreMesh(core_axis_name="c", subcore_axis_name="s")
    f = pl.kernel(
        _sc_body,
        out_type=(
            jax.ShapeDtypeStruct((R, K), jnp.int32),
            jax.ShapeDtypeStruct((R, K), jnp.int32),
        ),
        mesh=mesh,
        compiler_params=pltpu.CompilerParams(needs_layout_passes=False),
        scratch_types=[
            pltpu.VMEM((N,), jnp.int32),
            pltpu.VMEM((N,), jnp.int32),
            pltpu.VMEM((N + L,), jnp.int32),
            pltpu.VMEM((256 + L,), jnp.int32),
            pltpu.VMEM((K + L,), jnp.int32),
            pltpu.VMEM((K + L,), jnp.int32),
            pltpu.SemaphoreType.DMA,
            pltpu.SemaphoreType.DMA,
        ],
    )
    return f(xb)


def _roll_xor(x, pos, stride):
    bit = (pos & stride) == 0
    return jnp.where(bit, jnp.roll(x, -stride, axis=1),
                     jnp.roll(x, stride, axis=1))


def _tc_sort_body(key_ref, idx_ref, val_ref, ind_ref):
    keys = key_ref[...]
    idxs = idx_ref[...]
    pos = lax.broadcasted_iota(jnp.int32, (R, K), 1)
    size = 2
    while size <= K:
        stride = size // 2
        while stride >= 1:
            pk = _roll_xor(keys, pos, stride)
            pi = _roll_xor(idxs, pos, stride)
            a_pre_b = (keys > pk) | ((keys == pk) & (idxs < pi))
            second = (pos & stride) != 0
            flip = (pos & size) != 0
            take_a = a_pre_b ^ second ^ flip
            keys = jnp.where(take_a, keys, pk)
            idxs = jnp.where(take_a, idxs, pi)
            stride //= 2
        size *= 2
    s = jnp.where(keys >= 0, keys, keys ^ jnp.int32(0x7FFFFFFF))
    val_ref[...] = lax.bitcast_convert_type(s, jnp.float32)
    ind_ref[...] = idxs


def _tc_sort(okey, oidx):
    return pl.pallas_call(
        _tc_sort_body,
        out_shape=(
            jax.ShapeDtypeStruct((R, K), jnp.float32),
            jax.ShapeDtypeStruct((R, K), jnp.int32),
        ),
    )(okey, oidx)


def kernel(x, k):
    xb = lax.bitcast_convert_type(x, jnp.int32)
    okey, oidx = _sc_select(xb)
    vals, inds = _tc_sort(okey, oidx)
    vals = vals + (jnp.asarray(k) - K).astype(vals.dtype)
    return vals, inds
